# baseline (speedup 1.0000x reference)
import functools

import jax
import jax.numpy as jnp
from jax.experimental import pallas as pl
from jax.experimental.pallas import tpu as pltpu

_B, _P, _C = 32, 8732, 21
_PBLK = 1024
_NJ = (_P + _PBLK - 1) // _PBLK
_EPS = 1e-7


def _tr(x, k):
    ident = jnp.eye(k, dtype=jnp.float32)
    return jax.lax.dot_general(ident, x, (((1,), (1,)), ((), ())),
                               preferred_element_type=jnp.float32)


def _body(lam_ref, c_ref, t_ref, ci_ref, lo_ref, ls_ref, li_ref,
          o_ab, o_lc, o_rc, o_ll, o_rl, o_wi, o_wl, o_wr):
    b = pl.program_id(0)
    j = pl.program_id(1)

    @pl.when((b == 0) & (j == 0))
    def _init():
        for o in (o_ab, o_lc, o_rc, o_ll, o_rl, o_wi, o_wl, o_wr):
            o[...] = jnp.zeros_like(o)

    lam = lam_ref[0, 0]
    c = _tr(c_ref[0], _C)
    t = _tr(t_ref[0], _C)
    ci = _tr(ci_ref[0], _C)

    lane = jax.lax.broadcasted_iota(jnp.int32, (1, _PBLK), 1)
    valid = (j * _PBLK + lane) < _P
    c = jnp.where(valid, c, 0.5)
    t = jnp.where(valid, t, 0.5)
    ci = jnp.where(valid, ci, 0.5)

    cpe = c + _EPS
    tpe = t + _EPS
    ins = ci + _EPS
    mixed = lam * c + (1.0 - lam) * t + _EPS
    lg_m = jnp.log(mixed)
    lg_i = jnp.log(ins)
    lg_c = jnp.log(cpe)
    lg_t = jnp.log(tpe)
    d_im = lg_i - lg_m
    ab = (ins - mixed) * d_im
    lc = cpe * (lg_c - lg_i)
    rc = tpe * (lg_t - lg_i)
    ab_r = jnp.sum(ab, axis=0, keepdims=True)
    lc_r = jnp.sum(lc, axis=0, keepdims=True)
    rc_r = jnp.sum(rc, axis=0, keepdims=True)

    cmax = jnp.max(c, axis=0, keepdims=True)
    tmax = jnp.max(t, axis=0, keepdims=True)
    left = cmax > c[0:1]
    right = tmax > t[0:1]
    wi = (left & right & valid).astype(jnp.float32)
    wl = (left & ~right & valid).astype(jnp.float32)
    wr = (right & ~left & valid).astype(jnp.float32)

    lo = _tr(lo_ref[0], 4)
    ls = _tr(ls_ref[0], 4)
    li = _tr(li_ref[0], 4)
    dl = jnp.where(valid, li - lo, 0.0)
    dr = jnp.where(valid, li - ls, 0.0)
    ll_r = jnp.sum(dl * dl, axis=0, keepdims=True)
    rl_r = jnp.sum(dr * dr, axis=0, keepdims=True)

    o_ab[...] += ab_r * wi
    o_lc[...] += lc_r * wl
    o_rc[...] += rc_r * wr
    o_ll[...] += ll_r * wl
    o_rl[...] += rl_r * wr
    o_wi[...] += wi
    o_wl[...] += wl
    o_wr[...] += wr


@functools.partial(jax.jit, static_argnames=())
def kernel(conf, conf_flip, loc, loc_flip, conf_shuffle, conf_interpolation,
           loc_shuffle, loc_interpolation, lam):
    del conf_flip, loc_flip
    half = _B // 2
    lam_s = jnp.reshape(lam.astype(jnp.float32), (1, 1))

    conf_spec = pl.BlockSpec((1, _PBLK, _C), lambda b, j: (b, j, 0))
    swap_spec = pl.BlockSpec((1, _PBLK, _C), lambda b, j: ((b + half) % _B, j, 0))
    loc_spec = pl.BlockSpec((1, _PBLK, 4), lambda b, j: (b, j, 0))
    lswap_spec = pl.BlockSpec((1, _PBLK, 4), lambda b, j: ((b + half) % _B, j, 0))
    acc_spec = pl.BlockSpec((1, _PBLK), lambda b, j: (0, 0))
    acc_ty = jax.ShapeDtypeStruct((1, _PBLK), jnp.float32)

    outs = pl.pallas_call(
        _body,
        grid=(_B, _NJ),
        in_specs=[
            pl.BlockSpec(memory_space=pltpu.SMEM),
            conf_spec, swap_spec, conf_spec,
            loc_spec, lswap_spec, loc_spec,
        ],
        out_specs=[acc_spec] * 8,
        out_shape=[acc_ty] * 8,
        compiler_params=pltpu.CompilerParams(
            dimension_semantics=("arbitrary", "arbitrary"),
        ),
    )(lam_s, conf, conf_shuffle, conf_interpolation,
      loc, loc_shuffle, loc_interpolation)

    s_ab, s_lc, s_rc, s_ll, s_rl, n_i, n_l, n_r = [jnp.sum(o) for o in outs]

    def mmean(s, n):
        return jnp.where(n > 0, s / jnp.maximum(n, 1.0), jnp.float32(0.0))

    total = (mmean(s_ab, n_i) * 0.5
             + mmean(s_lc, n_l) + mmean(s_ll, n_l) * 0.25
             + mmean(s_rc, n_r) + mmean(s_rl, n_r) * 0.25)
    return total


# speedup vs baseline: 11.9259x; 11.9259x over previous
import functools

import jax
import jax.numpy as jnp
from jax.experimental import pallas as pl
from jax.experimental.pallas import tpu as pltpu

_B, _P, _C = 32, 8732, 21
_PBLK = 2048
_NJ = (_P + _PBLK - 1) // _PBLK
_GB = 8
_NG = _B // _GB
_EPS = 1e-7


def _body(lam_ref, c_ref, t_ref, ci_ref, lo_ref, ls_ref, li_ref,
          o_ab, o_lc, o_rc, o_ll, o_rl, o_wi, o_wl, o_wr):
    g = pl.program_id(0)
    j = pl.program_id(1)

    @pl.when((g == 0) & (j == 0))
    def _init():
        for o in (o_ab, o_lc, o_rc, o_ll, o_rl, o_wi, o_wl, o_wr):
            o[...] = jnp.zeros_like(o)

    lam = lam_ref[0, 0]
    lane = jax.lax.broadcasted_iota(jnp.int32, (1, _PBLK), 1)
    valid = (j * _PBLK + lane) < _P
    v3 = valid[None]

    c = jnp.where(v3, c_ref[...], 0.5)
    t = jnp.where(v3, t_ref[...], 0.5)
    ci = jnp.where(v3, ci_ref[...], 0.5)

    cpe = c + _EPS
    tpe = t + _EPS
    ins = ci + _EPS
    mixed = lam * c + (1.0 - lam) * t + _EPS
    lg_m = jnp.log(mixed)
    lg_i = jnp.log(ins)
    lg_c = jnp.log(cpe)
    lg_t = jnp.log(tpe)
    d_im = lg_i - lg_m
    ab = (ins - mixed) * d_im
    lc = cpe * (lg_c - lg_i)
    rc = tpe * (lg_t - lg_i)
    ab_r = jnp.sum(ab, axis=0)
    lc_r = jnp.sum(lc, axis=0)
    rc_r = jnp.sum(rc, axis=0)

    cmax = jnp.max(c, axis=0)
    tmax = jnp.max(t, axis=0)
    left = cmax > c[0]
    right = tmax > t[0]
    wi = (left & right & valid).astype(jnp.float32)
    wl = (left & ~right & valid).astype(jnp.float32)
    wr = (right & ~left & valid).astype(jnp.float32)

    lo = lo_ref[...]
    ls = ls_ref[...]
    li = li_ref[...]
    dl = jnp.where(valid[:, None], li - lo, 0.0)
    dr = jnp.where(valid[:, None], li - ls, 0.0)
    ll_r = jnp.sum(dl * dl, axis=1)
    rl_r = jnp.sum(dr * dr, axis=1)

    o_ab[...] += ab_r * wi
    o_lc[...] += lc_r * wl
    o_rc[...] += rc_r * wr
    o_ll[...] += ll_r * wl
    o_rl[...] += rl_r * wr
    o_wi[...] += wi
    o_wl[...] += wl
    o_wr[...] += wr


@functools.partial(jax.jit, static_argnames=())
def kernel(conf, conf_flip, loc, loc_flip, conf_shuffle, conf_interpolation,
           loc_shuffle, loc_interpolation, lam):
    del conf_flip, loc_flip
    lam_s = jnp.reshape(lam.astype(jnp.float32), (1, 1))

    cT = jnp.transpose(conf, (2, 0, 1))
    tT = jnp.transpose(conf_shuffle, (2, 0, 1))
    iT = jnp.transpose(conf_interpolation, (2, 0, 1))
    loT = jnp.transpose(loc, (0, 2, 1))
    lsT = jnp.transpose(loc_shuffle, (0, 2, 1))
    liT = jnp.transpose(loc_interpolation, (0, 2, 1))

    half_g = (_B // 2) // _GB
    conf_spec = pl.BlockSpec((_C, _GB, _PBLK), lambda g, j: (0, g, j))
    swap_spec = pl.BlockSpec((_C, _GB, _PBLK),
                             lambda g, j: (0, (g + half_g) % _NG, j))
    loc_spec = pl.BlockSpec((_GB, 4, _PBLK), lambda g, j: (g, 0, j))
    lswap_spec = pl.BlockSpec((_GB, 4, _PBLK),
                              lambda g, j: ((g + half_g) % _NG, 0, j))
    acc_spec = pl.BlockSpec((_GB, _PBLK), lambda g, j: (0, 0))
    acc_ty = jax.ShapeDtypeStruct((_GB, _PBLK), jnp.float32)

    outs = pl.pallas_call(
        _body,
        grid=(_NG, _NJ),
        in_specs=[
            pl.BlockSpec(memory_space=pltpu.SMEM),
            conf_spec, swap_spec, conf_spec,
            loc_spec, lswap_spec, loc_spec,
        ],
        out_specs=[acc_spec] * 8,
        out_shape=[acc_ty] * 8,
        compiler_params=pltpu.CompilerParams(
            dimension_semantics=("arbitrary", "arbitrary"),
        ),
    )(lam_s, cT, tT, iT, loT, lsT, liT)

    s_ab, s_lc, s_rc, s_ll, s_rl, n_i, n_l, n_r = [jnp.sum(o) for o in outs]

    def mmean(s, n):
        return jnp.where(n > 0, s / jnp.maximum(n, 1.0), jnp.float32(0.0))

    total = (mmean(s_ab, n_i) * 0.5
             + mmean(s_lc, n_l) + mmean(s_ll, n_l) * 0.25
             + mmean(s_rc, n_r) + mmean(s_rl, n_r) * 0.25)
    return total


# speedup vs baseline: 16.6813x; 1.3987x over previous
import functools

import jax
import jax.numpy as jnp
from jax.experimental import pallas as pl
from jax.experimental.pallas import tpu as pltpu

_B, _P, _C = 32, 8732, 21
_PBLK = 2048
_NJ = (_P + _PBLK - 1) // _PBLK
_GB = 8
_NG = _B // _GB
_EPS = 1e-7


def _body(lam_ref, c_ref, t_ref, ci_ref, lo_ref, ls_ref, li_ref,
          out_ref,
          a_ab, a_lc, a_rc, a_ll, a_rl, a_wi, a_wl, a_wr):
    g = pl.program_id(0)
    j = pl.program_id(1)

    @pl.when((g == 0) & (j == 0))
    def _init():
        for a in (a_ab, a_lc, a_rc, a_ll, a_rl, a_wi, a_wl, a_wr):
            a[...] = jnp.zeros_like(a)

    lam = lam_ref[0, 0]
    lane = jax.lax.broadcasted_iota(jnp.int32, (1, _PBLK), 1)
    valid = (j * _PBLK + lane) < _P
    v3 = valid[None]

    c = jnp.where(v3, c_ref[...], 0.5)
    t = jnp.where(v3, t_ref[...], 0.5)
    ci = jnp.where(v3, ci_ref[...], 0.5)

    cpe = c + _EPS
    tpe = t + _EPS
    ins = ci + _EPS
    mixed = lam * c + (1.0 - lam) * t + _EPS
    inv_i = 1.0 / ins
    inv_m = 1.0 / mixed
    d_im = jnp.log(ins * inv_m)
    ab = (ins - mixed) * d_im
    lc = cpe * jnp.log(cpe * inv_i)
    rc = tpe * jnp.log(tpe * inv_i)
    ab_r = jnp.sum(ab, axis=0)
    lc_r = jnp.sum(lc, axis=0)
    rc_r = jnp.sum(rc, axis=0)

    cmax = jnp.max(c, axis=0)
    tmax = jnp.max(t, axis=0)
    left = cmax > c[0]
    right = tmax > t[0]
    wi = (left & right & valid).astype(jnp.float32)
    wl = (left & ~right & valid).astype(jnp.float32)
    wr = (right & ~left & valid).astype(jnp.float32)

    lo = lo_ref[...]
    ls = ls_ref[...]
    li = li_ref[...]
    dl = jnp.where(valid[:, None], li - lo, 0.0)
    dr = jnp.where(valid[:, None], li - ls, 0.0)
    ll_r = jnp.sum(dl * dl, axis=1)
    rl_r = jnp.sum(dr * dr, axis=1)

    a_ab[...] += ab_r * wi
    a_lc[...] += lc_r * wl
    a_rc[...] += rc_r * wr
    a_ll[...] += ll_r * wl
    a_rl[...] += rl_r * wr
    a_wi[...] += wi
    a_wl[...] += wl
    a_wr[...] += wr

    @pl.when((g == _NG - 1) & (j == _NJ - 1))
    def _final():
        s_ab = jnp.sum(a_ab[...])
        s_lc = jnp.sum(a_lc[...])
        s_rc = jnp.sum(a_rc[...])
        s_ll = jnp.sum(a_ll[...])
        s_rl = jnp.sum(a_rl[...])
        n_i = jnp.sum(a_wi[...])
        n_l = jnp.sum(a_wl[...])
        n_r = jnp.sum(a_wr[...])

        def mmean(s, n):
            return jnp.where(n > 0, s / jnp.maximum(n, 1.0), jnp.float32(0.0))

        out_ref[0, 0] = (mmean(s_ab, n_i) * 0.5
                         + mmean(s_lc, n_l) + mmean(s_ll, n_l) * 0.25
                         + mmean(s_rc, n_r) + mmean(s_rl, n_r) * 0.25)


@functools.partial(jax.jit, static_argnames=())
def kernel(conf, conf_flip, loc, loc_flip, conf_shuffle, conf_interpolation,
           loc_shuffle, loc_interpolation, lam):
    del conf_flip, loc_flip
    lam_s = jnp.reshape(lam.astype(jnp.float32), (1, 1))

    cT = jnp.transpose(conf, (2, 0, 1))
    tT = jnp.transpose(conf_shuffle, (2, 0, 1))
    iT = jnp.transpose(conf_interpolation, (2, 0, 1))
    loT = jnp.transpose(loc, (0, 2, 1))
    lsT = jnp.transpose(loc_shuffle, (0, 2, 1))
    liT = jnp.transpose(loc_interpolation, (0, 2, 1))

    half_g = (_B // 2) // _GB
    conf_spec = pl.BlockSpec((_C, _GB, _PBLK), lambda g, j: (0, g, j))
    swap_spec = pl.BlockSpec((_C, _GB, _PBLK),
                             lambda g, j: (0, (g + half_g) % _NG, j))
    loc_spec = pl.BlockSpec((_GB, 4, _PBLK), lambda g, j: (g, 0, j))
    lswap_spec = pl.BlockSpec((_GB, 4, _PBLK),
                              lambda g, j: ((g + half_g) % _NG, 0, j))

    out = pl.pallas_call(
        _body,
        grid=(_NG, _NJ),
        in_specs=[
            pl.BlockSpec(memory_space=pltpu.SMEM),
            conf_spec, swap_spec, conf_spec,
            loc_spec, lswap_spec, loc_spec,
        ],
        out_specs=pl.BlockSpec(memory_space=pltpu.SMEM),
        out_shape=jax.ShapeDtypeStruct((1, 1), jnp.float32),
        scratch_shapes=[pltpu.VMEM((_GB, _PBLK), jnp.float32)] * 8,
        compiler_params=pltpu.CompilerParams(
            dimension_semantics=("arbitrary", "arbitrary"),
        ),
    )(lam_s, cT, tT, iT, loT, lsT, liT)

    return out[0, 0]


# speedup vs baseline: 20.4215x; 1.2242x over previous
import functools

import jax
import jax.numpy as jnp
from jax.experimental import pallas as pl
from jax.experimental.pallas import tpu as pltpu

_B, _P, _C = 32, 8732, 21
_PBLK = 8832
_NJ = (_P + _PBLK - 1) // _PBLK
_GB = 8
_NG = _B // _GB
_EPS = 1e-7


def _body(lam_ref, c_ref, t_ref, ci_ref, lo_ref, ls_ref, li_ref,
          out_ref,
          a_ab, a_lc, a_rc, a_ll, a_rl, a_wi, a_wl, a_wr):
    g = pl.program_id(0)
    j = pl.program_id(1)

    @pl.when((g == 0) & (j == 0))
    def _init():
        for a in (a_ab, a_lc, a_rc, a_ll, a_rl, a_wi, a_wl, a_wr):
            a[...] = jnp.zeros_like(a)

    lam = lam_ref[0, 0]
    lane = jax.lax.broadcasted_iota(jnp.int32, (1, _PBLK), 1)
    valid = (j * _PBLK + lane) < _P
    v3 = valid[None]

    c = jnp.where(v3, c_ref[...], 0.5)
    t = jnp.where(v3, t_ref[...], 0.5)
    ci = jnp.where(v3, ci_ref[...], 0.5)

    cpe = c + _EPS
    tpe = t + _EPS
    ins = ci + _EPS
    mixed = lam * c + (1.0 - lam) * t + _EPS
    inv_i = 1.0 / ins
    inv_m = 1.0 / mixed
    d_im = jnp.log(ins * inv_m)
    ab = (ins - mixed) * d_im
    lc = cpe * jnp.log(cpe * inv_i)
    rc = tpe * jnp.log(tpe * inv_i)
    ab_r = jnp.sum(ab, axis=0)
    lc_r = jnp.sum(lc, axis=0)
    rc_r = jnp.sum(rc, axis=0)

    cmax = jnp.max(c, axis=0)
    tmax = jnp.max(t, axis=0)
    left = cmax > c[0]
    right = tmax > t[0]
    wi = (left & right & valid).astype(jnp.float32)
    wl = (left & ~right & valid).astype(jnp.float32)
    wr = (right & ~left & valid).astype(jnp.float32)

    lo = lo_ref[...]
    ls = ls_ref[...]
    li = li_ref[...]
    dl = jnp.where(valid[:, None], li - lo, 0.0)
    dr = jnp.where(valid[:, None], li - ls, 0.0)
    ll_r = jnp.sum(dl * dl, axis=1)
    rl_r = jnp.sum(dr * dr, axis=1)

    a_ab[...] += ab_r * wi
    a_lc[...] += lc_r * wl
    a_rc[...] += rc_r * wr
    a_ll[...] += ll_r * wl
    a_rl[...] += rl_r * wr
    a_wi[...] += wi
    a_wl[...] += wl
    a_wr[...] += wr

    @pl.when((g == _NG - 1) & (j == _NJ - 1))
    def _final():
        s_ab = jnp.sum(a_ab[...])
        s_lc = jnp.sum(a_lc[...])
        s_rc = jnp.sum(a_rc[...])
        s_ll = jnp.sum(a_ll[...])
        s_rl = jnp.sum(a_rl[...])
        n_i = jnp.sum(a_wi[...])
        n_l = jnp.sum(a_wl[...])
        n_r = jnp.sum(a_wr[...])

        def mmean(s, n):
            return jnp.where(n > 0, s / jnp.maximum(n, 1.0), jnp.float32(0.0))

        out_ref[0, 0] = (mmean(s_ab, n_i) * 0.5
                         + mmean(s_lc, n_l) + mmean(s_ll, n_l) * 0.25
                         + mmean(s_rc, n_r) + mmean(s_rl, n_r) * 0.25)


@functools.partial(jax.jit, static_argnames=())
def kernel(conf, conf_flip, loc, loc_flip, conf_shuffle, conf_interpolation,
           loc_shuffle, loc_interpolation, lam):
    del conf_flip, loc_flip
    lam_s = jnp.reshape(lam.astype(jnp.float32), (1, 1))

    cT = jnp.transpose(conf, (2, 0, 1))
    tT = jnp.transpose(conf_shuffle, (2, 0, 1))
    iT = jnp.transpose(conf_interpolation, (2, 0, 1))
    loT = jnp.transpose(loc, (0, 2, 1))
    lsT = jnp.transpose(loc_shuffle, (0, 2, 1))
    liT = jnp.transpose(loc_interpolation, (0, 2, 1))

    half_g = (_B // 2) // _GB
    conf_spec = pl.BlockSpec((_C, _GB, _PBLK), lambda g, j: (0, g, j))
    swap_spec = pl.BlockSpec((_C, _GB, _PBLK),
                             lambda g, j: (0, (g + half_g) % _NG, j))
    loc_spec = pl.BlockSpec((_GB, 4, _PBLK), lambda g, j: (g, 0, j))
    lswap_spec = pl.BlockSpec((_GB, 4, _PBLK),
                              lambda g, j: ((g + half_g) % _NG, 0, j))

    out = pl.pallas_call(
        _body,
        grid=(_NG, _NJ),
        in_specs=[
            pl.BlockSpec(memory_space=pltpu.SMEM),
            conf_spec, swap_spec, conf_spec,
            loc_spec, lswap_spec, loc_spec,
        ],
        out_specs=pl.BlockSpec(memory_space=pltpu.SMEM),
        out_shape=jax.ShapeDtypeStruct((1, 1), jnp.float32),
        scratch_shapes=[pltpu.VMEM((_GB, _PBLK), jnp.float32)] * 8,
        compiler_params=pltpu.CompilerParams(
            dimension_semantics=("arbitrary", "arbitrary"),
        ),
    )(lam_s, cT, tT, iT, loT, lsT, liT)

    return out[0, 0]
